# baseline (speedup 1.0000x reference)
import functools

import jax
import jax.numpy as jnp
from jax import lax
from jax.experimental import pallas as pl
from jax.experimental.pallas import tpu as pltpu
from jax.experimental.pallas import tpu_sc as plsc

_B = 8
_MAX_SEQLEN = 2048
_D = 512
_TOTAL = 8192

_NC = 2
_NS = 16
_NW = _NC * _NS
_ROWS = _B * _MAX_SEQLEN
_ROWS_PER_W = _ROWS // _NW
_W_PER_B = _MAX_SEQLEN // _ROWS_PER_W
_CHUNK = 128
_NCHUNK = _ROWS_PER_W // _CHUNK
_ZROWS = 16


def _body(flat_hbm, cu_hbm, out_hbm, cu_v, zeros_v):
    wid = lax.axis_index("s") * _NC + lax.axis_index("c")
    b = wid // _W_PER_B
    base_s = (wid % _W_PER_B) * _ROWS_PER_W

    def _zrow(j, carry):
        for i in range(_ZROWS):
            zeros_v[i, pl.ds(j * 16, 16)] = jnp.zeros((16,), jnp.float32)
        return carry

    lax.fori_loop(0, _D // 16, _zrow, 0)

    pltpu.sync_copy(cu_hbm, cu_v)
    cu_vec = cu_v[...]
    vals = [
        lax.squeeze(lax.slice(cu_vec, (i,), (i + 1,)), (0,))
        for i in range(_B + 1)
    ]
    cu_b = jnp.int32(0)
    cu_b1 = jnp.int32(0)
    for i in range(_B + 1):
        cu_b = jnp.where(b == i, vals[i], cu_b)
        cu_b1 = jnp.where(b + 1 == i, vals[i], cu_b1)
    seg_len = cu_b1 - cu_b

    k = jnp.clip(seg_len - base_s, 0, _ROWS_PER_W)

    for c in range(_NCHUNK):
        n_c = jnp.clip(k - c * _CHUNK, 0, _CHUNK)
        src = cu_b + base_s + c * _CHUNK
        dst = wid * _ROWS_PER_W + c * _CHUNK

        @pl.when(n_c == _CHUNK)
        def _full():
            pltpu.sync_copy(
                flat_hbm.at[pl.ds(src, _CHUNK)],
                out_hbm.at[pl.ds(dst, _CHUNK)],
            )

        @pl.when(n_c == 0)
        def _zero():
            for z in range(_CHUNK // _ZROWS):
                pltpu.sync_copy(
                    zeros_v,
                    out_hbm.at[pl.ds(dst + z * _ZROWS, _ZROWS)],
                )

        @pl.when(jnp.logical_and(n_c > 0, n_c < _CHUNK))
        def _partial():
            def _row(r, carry):
                @pl.when(r < n_c)
                def _copy():
                    pltpu.sync_copy(
                        flat_hbm.at[pl.ds(src + r, 1)],
                        out_hbm.at[pl.ds(dst + r, 1)],
                    )

                @pl.when(r >= n_c)
                def _pad():
                    pltpu.sync_copy(
                        zeros_v.at[pl.ds(0, 1)],
                        out_hbm.at[pl.ds(dst + r, 1)],
                    )

                return carry

            lax.fori_loop(0, _CHUNK, _row, 0)


@jax.jit
def kernel(flat, cu_seqlens):
    cu16 = jnp.zeros((16,), jnp.int32).at[: _B + 1].set(cu_seqlens)
    run = functools.partial(
        pl.kernel,
        mesh=plsc.VectorSubcoreMesh(core_axis_name="c", subcore_axis_name="s"),
        out_type=jax.ShapeDtypeStruct((_ROWS, _D), jnp.float32),
        scratch_types=[
            pltpu.VMEM((16,), jnp.int32),
            pltpu.VMEM((_ZROWS, _D), jnp.float32),
        ],
        compiler_params=pltpu.CompilerParams(use_tc_tiling_on_sc=False),
    )(_body)
    dense = run(flat, cu16)
    return dense.reshape(_B, _MAX_SEQLEN, _D)
